# baseline (speedup 1.0000x reference)
import functools

import jax
import jax.numpy as jnp
from jax import lax
from jax.experimental import pallas as pl
from jax.experimental.pallas import tpu as pltpu
from jax.experimental.pallas import tpu_sc as plsc

N = 10000
NPAD = 10240
F = 128
E = 320000
NC = 2
NS = 16
NW = NC * NS
EPW = E // NW
CHUNK = 80
NCHUNK = EPW // CHUNK
RPT = NPAD // NS

_mesh = plsc.VectorSubcoreMesh(core_axis_name="c", subcore_axis_name="s")


@functools.partial(
    pl.kernel,
    mesh=_mesh,
    out_type=jax.ShapeDtypeStruct((NW, N), jnp.float32),
    scratch_types=[
        pltpu.VMEM((EPW,), jnp.int32),
        pltpu.VMEM((N,), jnp.float32),
    ],
    compiler_params=pltpu.CompilerParams(needs_layout_passes=False),
)
def _deg_kernel(dst_hbm, out_hbm, dst_v, deg_v):
    c = lax.axis_index("c")
    s = lax.axis_index("s")
    wid = s * NC + c
    pltpu.sync_copy(dst_hbm.at[pl.ds(wid * EPW, EPW)], dst_v)

    def zero_body(i, carry):
        deg_v[pl.ds(i * 16, 16)] = jnp.zeros((16,), jnp.float32)
        return carry

    lax.fori_loop(0, N // 16, zero_body, 0)

    ones = jnp.ones((16,), jnp.float32)

    def acc_body(i, carry):
        idx = dst_v[pl.ds(i * 16, 16)]
        plsc.addupdate_scatter(deg_v, [idx], ones)
        return carry

    lax.fori_loop(0, EPW // 16, acc_body, 0)

    pltpu.sync_copy(deg_v, out_hbm.at[wid])


@functools.partial(
    pl.kernel,
    mesh=_mesh,
    out_type=jax.ShapeDtypeStruct((2 * NPAD, F), jnp.float32),
    scratch_types=[
        pltpu.VMEM((EPW,), jnp.int32),
        pltpu.VMEM((NCHUNK, CHUNK), jnp.int32),
        pltpu.VMEM((CHUNK, F), jnp.float32),
        pltpu.VMEM((CHUNK, F), jnp.float32),
        pltpu.VMEM_SHARED((NPAD, F), jnp.float32),
        pltpu.SemaphoreType.DMA,
        pltpu.SemaphoreType.DMA,
        pltpu.SemaphoreType.DMA,
        pltpu.SemaphoreType.DMA,
    ],
)
def _agg_kernel(g_hbm, src_hbm, dstr_hbm, zeros_hbm, out_hbm,
                src_v, dst_v, r0, r1, acc_sh, gsA, gsB, ssA, ssB):
    c = lax.axis_index("c")
    s = lax.axis_index("s")
    wid = s * NC + c
    pltpu.sync_copy(src_hbm.at[pl.ds(wid * EPW, EPW)], src_v)
    pltpu.sync_copy(dstr_hbm.at[wid], dst_v)
    pltpu.sync_copy(zeros_hbm.at[pl.ds(s * RPT, RPT)],
                    acc_sh.at[pl.ds(s * RPT, RPT)])
    plsc.subcore_barrier()

    def start_gather(i, buf, sem):
        pltpu.async_copy(g_hbm.at[src_v.at[pl.ds(i * CHUNK, CHUNK)]],
                         buf, sem)

    def wait_gather(i, buf, sem):
        pltpu.make_async_copy(g_hbm.at[src_v.at[pl.ds(i * CHUNK, CHUNK)]],
                              buf, sem).wait()

    def start_scatter(i, buf, sem):
        pltpu.async_copy(buf, acc_sh.at[dst_v.at[i]], sem, add=True)

    def wait_scatter(i, buf, sem):
        pltpu.make_async_copy(buf, acc_sh.at[dst_v.at[i]], sem).wait()

    start_gather(0, r0, gsA)

    def pair(p, carry):
        i0 = 2 * p

        wait_gather(i0, r0, gsA)

        @pl.when(p > 0)
        def _():
            wait_scatter(i0 - 1, r1, ssB)

        start_gather(i0 + 1, r1, gsB)
        start_scatter(i0, r0, ssA)

        wait_gather(i0 + 1, r1, gsB)
        wait_scatter(i0, r0, ssA)
        start_gather(i0 + 2, r0, gsA)
        start_scatter(i0 + 1, r1, ssB)
        return carry

    lax.fori_loop(0, (NCHUNK - 1) // 2, pair, 0)

    wait_gather(NCHUNK - 1, r0, gsA)
    wait_scatter(NCHUNK - 2, r1, ssB)
    start_scatter(NCHUNK - 1, r0, ssA)
    wait_scatter(NCHUNK - 1, r0, ssA)

    plsc.subcore_barrier()
    pltpu.sync_copy(acc_sh.at[pl.ds(s * RPT, RPT)],
                    out_hbm.at[pl.ds(c * NPAD + s * RPT, RPT)])


def _dinv_from_partials(degp):
    return lax.rsqrt(jnp.sum(degp, axis=0) + 1.0)


def _tc1_body(x_ref, w1_ref, degp_ref, g1_ref):
    dinv = _dinv_from_partials(degp_ref[...])
    h = jnp.dot(x_ref[...], w1_ref[...], preferred_element_type=jnp.float32)
    g1_ref[...] = h * dinv[:, None]


def _tc2_body(p_ref, g1_ref, degp_ref, b1_ref, w2_ref, g2_ref):
    dinv = _dinv_from_partials(degp_ref[...])
    p = p_ref[...]
    g1 = g1_ref[...]
    agg = p[:N] + p[NPAD:NPAD + N] + g1
    out1 = jnp.maximum(agg * dinv[:, None] + b1_ref[...][None, :], 0.0)
    h2 = jnp.dot(out1, w2_ref[...], preferred_element_type=jnp.float32)
    g2_ref[...] = h2 * dinv[:, None]


def _tc3_body(p_ref, g2_ref, degp_ref, b2_ref, wh_ref, bh_ref,
              scores_ref, h_ref):
    dinv = _dinv_from_partials(degp_ref[...])
    p = p_ref[...]
    g2 = g2_ref[...]
    agg = p[:N] + p[NPAD:NPAD + N] + g2
    out2 = jnp.maximum(agg * dinv[:, None] + b2_ref[...][None, :], 0.0)
    h_ref[...] = out2
    scores_ref[...] = (
        jnp.dot(out2, wh_ref[...], preferred_element_type=jnp.float32)
        + bh_ref[...][None, :]
    )


_tc1 = pl.pallas_call(
    _tc1_body,
    out_shape=jax.ShapeDtypeStruct((N, F), jnp.float32),
)

_tc2 = pl.pallas_call(
    _tc2_body,
    out_shape=jax.ShapeDtypeStruct((N, F), jnp.float32),
)

_tc3 = pl.pallas_call(
    _tc3_body,
    out_shape=(
        jax.ShapeDtypeStruct((N, 40), jnp.float32),
        jax.ShapeDtypeStruct((N, F), jnp.float32),
    ),
)


def kernel(x, edge_index, W1, b1, W2, b2, Wh, bh):
    dstr = edge_index[1].reshape(NW, NCHUNK, CHUNK)
    zeros_nf = jnp.zeros((NPAD, F), jnp.float32)

    degp = _deg_kernel(edge_index[1])
    g1 = _tc1(x, W1, degp)
    p1 = _agg_kernel(g1, edge_index[0], dstr, zeros_nf)
    g2 = _tc2(p1, g1, degp, b1, W2)
    p2 = _agg_kernel(g2, edge_index[0], dstr, zeros_nf)
    scores, h = _tc3(p2, g2, degp, b2, Wh, bh)
    return (scores, h)


# speedup vs baseline: 1.0163x; 1.0163x over previous
import functools

import jax
import jax.numpy as jnp
from jax import lax
from jax.experimental import pallas as pl
from jax.experimental.pallas import tpu as pltpu
from jax.experimental.pallas import tpu_sc as plsc

N = 10000
NPAD = 10240
F = 128
E = 320000
NC = 2
NS = 16
NW = NC * NS
EPW = E // NW
CHUNK = 80
NCHUNK = EPW // CHUNK
RPT = NPAD // NS

_mesh = plsc.VectorSubcoreMesh(core_axis_name="c", subcore_axis_name="s")


@functools.partial(
    pl.kernel,
    mesh=_mesh,
    out_type=jax.ShapeDtypeStruct((NW, N), jnp.float32),
    scratch_types=[
        pltpu.VMEM((EPW,), jnp.int32),
        pltpu.VMEM((N,), jnp.float32),
    ],
    compiler_params=pltpu.CompilerParams(needs_layout_passes=False),
)
def _deg_kernel(dst_hbm, out_hbm, dst_v, deg_v):
    c = lax.axis_index("c")
    s = lax.axis_index("s")
    wid = s * NC + c
    pltpu.sync_copy(dst_hbm.at[pl.ds(wid * EPW, EPW)], dst_v)

    def zero_body(i, carry):
        deg_v[pl.ds(i * 16, 16)] = jnp.zeros((16,), jnp.float32)
        return carry

    lax.fori_loop(0, N // 16, zero_body, 0)

    ones = jnp.ones((16,), jnp.float32)

    def acc_body(i, carry):
        idx = dst_v[pl.ds(i * 16, 16)]
        plsc.addupdate_scatter(deg_v, [idx], ones)
        return carry

    lax.fori_loop(0, EPW // 16, acc_body, 0)

    pltpu.sync_copy(deg_v, out_hbm.at[wid])


@functools.partial(
    pl.kernel,
    mesh=_mesh,
    out_type=jax.ShapeDtypeStruct((2 * NPAD, F), jnp.float32),
    scratch_types=[
        pltpu.VMEM((EPW,), jnp.int32),
        pltpu.VMEM((NCHUNK, CHUNK), jnp.int32),
        pltpu.VMEM((CHUNK, F), jnp.float32),
        pltpu.VMEM((CHUNK, F), jnp.float32),
        pltpu.VMEM_SHARED((NPAD, F), jnp.float32),
        pltpu.SemaphoreType.DMA,
        pltpu.SemaphoreType.DMA,
        pltpu.SemaphoreType.DMA,
        pltpu.SemaphoreType.DMA,
    ],
)
def _agg_kernel(g_hbm, src_hbm, dstr_hbm, out_hbm,
                src_v, dst_v, r0, r1, acc_sh, gsA, gsB, ssA, ssB):
    c = lax.axis_index("c")
    s = lax.axis_index("s")
    wid = s * NC + c
    pltpu.sync_copy(src_hbm.at[pl.ds(wid * EPW, EPW)], src_v)
    pltpu.sync_copy(dstr_hbm.at[wid], dst_v)
    r0[...] = jnp.zeros((CHUNK, F), jnp.float32)
    for z in range(RPT // CHUNK):
        pltpu.sync_copy(r0, acc_sh.at[pl.ds(s * RPT + z * CHUNK, CHUNK)])
    plsc.subcore_barrier()

    def start_gather(i, buf, sem):
        pltpu.async_copy(g_hbm.at[src_v.at[pl.ds(i * CHUNK, CHUNK)]],
                         buf, sem)

    def wait_gather(i, buf, sem):
        pltpu.make_async_copy(g_hbm.at[src_v.at[pl.ds(i * CHUNK, CHUNK)]],
                              buf, sem).wait()

    def start_scatter(i, buf, sem):
        pltpu.async_copy(buf, acc_sh.at[dst_v.at[i]], sem, add=True)

    def wait_scatter(i, buf, sem):
        pltpu.make_async_copy(buf, acc_sh.at[dst_v.at[i]], sem).wait()

    start_gather(0, r0, gsA)

    def pair(p, carry):
        i0 = 2 * p

        wait_gather(i0, r0, gsA)

        @pl.when(p > 0)
        def _():
            wait_scatter(i0 - 1, r1, ssB)

        start_gather(i0 + 1, r1, gsB)
        start_scatter(i0, r0, ssA)

        wait_gather(i0 + 1, r1, gsB)
        wait_scatter(i0, r0, ssA)
        start_gather(i0 + 2, r0, gsA)
        start_scatter(i0 + 1, r1, ssB)
        return carry

    lax.fori_loop(0, (NCHUNK - 1) // 2, pair, 0)

    wait_gather(NCHUNK - 1, r0, gsA)
    wait_scatter(NCHUNK - 2, r1, ssB)
    start_scatter(NCHUNK - 1, r0, ssA)
    wait_scatter(NCHUNK - 1, r0, ssA)

    plsc.subcore_barrier()
    pltpu.sync_copy(acc_sh.at[pl.ds(s * RPT, RPT)],
                    out_hbm.at[pl.ds(c * NPAD + s * RPT, RPT)])


def _dinv_from_partials(degp):
    return lax.rsqrt(jnp.sum(degp, axis=0) + 1.0)


def _tc1_body(x_ref, w1_ref, degp_ref, g1_ref):
    dinv = _dinv_from_partials(degp_ref[...])
    h = jnp.dot(x_ref[...], w1_ref[...], preferred_element_type=jnp.float32)
    g1_ref[...] = h * dinv[:, None]


def _tc2_body(p_ref, g1_ref, degp_ref, b1_ref, w2_ref, g2_ref):
    dinv = _dinv_from_partials(degp_ref[...])
    p = p_ref[...]
    g1 = g1_ref[...]
    agg = p[:N] + p[NPAD:NPAD + N] + g1
    out1 = jnp.maximum(agg * dinv[:, None] + b1_ref[...][None, :], 0.0)
    h2 = jnp.dot(out1, w2_ref[...], preferred_element_type=jnp.float32)
    g2_ref[...] = h2 * dinv[:, None]


def _tc3_body(p_ref, g2_ref, degp_ref, b2_ref, wh_ref, bh_ref,
              scores_ref, h_ref):
    dinv = _dinv_from_partials(degp_ref[...])
    p = p_ref[...]
    g2 = g2_ref[...]
    agg = p[:N] + p[NPAD:NPAD + N] + g2
    out2 = jnp.maximum(agg * dinv[:, None] + b2_ref[...][None, :], 0.0)
    h_ref[...] = out2
    scores_ref[...] = (
        jnp.dot(out2, wh_ref[...], preferred_element_type=jnp.float32)
        + bh_ref[...][None, :]
    )


_tc1 = pl.pallas_call(
    _tc1_body,
    out_shape=jax.ShapeDtypeStruct((N, F), jnp.float32),
)

_tc2 = pl.pallas_call(
    _tc2_body,
    out_shape=jax.ShapeDtypeStruct((N, F), jnp.float32),
)

_tc3 = pl.pallas_call(
    _tc3_body,
    out_shape=(
        jax.ShapeDtypeStruct((N, 40), jnp.float32),
        jax.ShapeDtypeStruct((N, F), jnp.float32),
    ),
)


def kernel(x, edge_index, W1, b1, W2, b2, Wh, bh):
    dstr = edge_index[1].reshape(NW, NCHUNK, CHUNK)

    degp = _deg_kernel(edge_index[1])
    g1 = _tc1(x, W1, degp)
    p1 = _agg_kernel(g1, edge_index[0], dstr)
    g2 = _tc2(p1, g1, degp, b1, W2)
    p2 = _agg_kernel(g2, edge_index[0], dstr)
    scores, h = _tc3(p2, g2, degp, b2, Wh, bh)
    return (scores, h)
